# baseline (speedup 1.0000x reference)
import functools

import jax
import jax.numpy as jnp
from jax import lax
from jax.experimental import pallas as pl
from jax.experimental.pallas import tpu as pltpu
from jax.experimental.pallas import tpu_sc as plsc

N = 50000
E = 800000
HID = 64
EMB = 16
B = 32
T_BUCKETS = 64
C_BUCKETS = 2048

NC = 2
NS = 16
NW = NC * NS

N_PAD = 50176
NPW = N_PAD // NW
NPT = N_PAD // NS
EC = 112
ECE = 416
NCHE = 124
EPT = ECE * NCHE
E_PAD = EPT * NS
EPW = E_PAD // NW
R = 512
GRID = N_PAD // R


def _mesh():
    return plsc.VectorSubcoreMesh(core_axis_name="c", subcore_axis_name="s")


_SC_PARAMS = pltpu.CompilerParams(use_tc_tiling_on_sc=False)


EMBC = NPW // 2

@functools.partial(
    pl.kernel,
    out_type=[jax.ShapeDtypeStruct((N_PAD, EMB), jnp.float32)] * 6,
    mesh=_mesh(),
    compiler_params=_SC_PARAMS,
    scratch_types=(
        [pltpu.VMEM((EMBC,), jnp.int32)] * 4
        + [pltpu.VMEM((EMBC, EMB), jnp.float32)] * 4
        + [pltpu.SemaphoreType.DMA] * 4
        + [pltpu.VMEM((ECE,), jnp.int32),
           pltpu.VMEM((ECE, EMB), jnp.float32),
           pltpu.VMEM_SHARED((N_PAD, EMB), jnp.float32)]
    ),
)
def _embdeg_kernel(ti, ci, di, xi, te, ce, de, xe, dste, ones_h, zeros_h,
                   ot, oc, od, ox, dega, degb,
                   i0, i1, i2, i3, r0, r1, r2, r3, m0, m1, m2, m3,
                   ei_v, ones_v, acc):
    c = lax.axis_index("c")
    s = lax.axis_index("s")
    wid = c * NS + s
    base = wid * NPW
    lanes = ((ti, te, ot, i0, r0, m0), (ci, ce, oc, i1, r1, m1),
             (di, de, od, i2, r2, m2), (xi, xe, ox, i3, r3, m3))

    def fire_emb(h):
        off = base + h * EMBC
        for iref, tab, out, iv, rv, sem in lanes:
            pltpu.sync_copy(iref.at[pl.ds(off, EMBC)], iv)
            pltpu.async_copy(tab.at[iv], rv, sem)

    def drain_emb(h):
        off = base + h * EMBC
        for iref, tab, out, iv, rv, sem in lanes:
            pltpu.make_async_copy(tab.at[iv], rv, sem).wait()
            pltpu.sync_copy(rv, out.at[pl.ds(off, EMBC)])

    pltpu.sync_copy(zeros_h, acc.at[pl.ds(s * NPT, NPT)])
    pltpu.sync_copy(ones_h, ones_v)
    fire_emb(0)
    plsc.subcore_barrier()
    ebase = (c * NS + s) * EPW

    def chunk(j, carry):
        off = ebase + j * ECE
        pltpu.sync_copy(dste.at[pl.ds(off, ECE)], ei_v)
        pltpu.sync_copy(ones_v, acc.at[ei_v], add=True)
        return carry

    lax.fori_loop(0, EPW // ECE // 2, chunk, 0)
    drain_emb(0)
    fire_emb(1)
    lax.fori_loop(EPW // ECE // 2, EPW // ECE, chunk, 0)
    drain_emb(1)
    plsc.subcore_barrier()
    nb = s * NPT

    @pl.when(c == 0)
    def _():
        pltpu.sync_copy(acc.at[pl.ds(nb, NPT)], dega.at[pl.ds(nb, NPT)])

    @pl.when(c == 1)
    def _():
        pltpu.sync_copy(acc.at[pl.ds(nb, NPT)], degb.at[pl.ds(nb, NPT)])


@functools.partial(
    pl.kernel,
    out_type=[jax.ShapeDtypeStruct((N_PAD, 32), jnp.float32)] * 2,
    mesh=_mesh(),
    compiler_params=_SC_PARAMS,
    scratch_types=(
        [pltpu.VMEM((ECE,), jnp.int32)] * 4
        + [pltpu.VMEM((ECE, 32), jnp.float32)] * 2
        + [pltpu.VMEM_SHARED((N_PAD, 32), jnp.float32)]
        + [pltpu.SemaphoreType.DMA] * 2
    ),
)
def _agg_kernel(glo, ghi, src, dst, zeros_h, mlo, mhi,
                s0, s1, d0, d1, r0, r1, acc, semA, semB):
    c = lax.axis_index("c")
    s = lax.axis_index("s")
    pltpu.sync_copy(zeros_h, acc.at[pl.ds(s * NPT, NPT)])
    plsc.subcore_barrier()
    ebase = s * EPT

    def run(g):
        bufs = ((s0, d0, r0, semA), (s1, d1, r1, semB))

        def fire(bi, j):
            sv, dv, rv, sem = bufs[bi]
            off = ebase + j * ECE
            pltpu.sync_copy(src.at[pl.ds(off, ECE)], sv)
            pltpu.sync_copy(dst.at[pl.ds(off, ECE)], dv)
            pltpu.async_copy(g.at[sv], rv, sem)

        def drain_scatter(bi):
            sv, dv, rv, sem = bufs[bi]
            pltpu.make_async_copy(g.at[sv], rv, sem).wait()
            pltpu.sync_copy(rv, acc.at[dv], add=True)

        fire(0, 0)

        def body(jj, carry):
            j0 = jj * 2
            for b in (0, 1):
                nxt = j0 + b + 1

                @pl.when(nxt < NCHE)
                def _():
                    fire((b + 1) % 2, nxt)

                drain_scatter(b)
            return carry

        lax.fori_loop(0, NCHE // 2, body, 0)

    @pl.when(c == 0)
    def _():
        run(glo)

    @pl.when(c == 1)
    def _():
        run(ghi)

    plsc.subcore_barrier()
    nb = s * NPT

    @pl.when(c == 0)
    def _():
        pltpu.sync_copy(acc.at[pl.ds(nb, NPT)], mlo.at[pl.ds(nb, NPT)])

    @pl.when(c == 1)
    def _():
        pltpu.sync_copy(acc.at[pl.ds(nb, NPT)], mhi.at[pl.ds(nb, NPT)])


def _scale0_body(da, db, xt, xc, xd, xx, dv_o, glo_o, ghi_o):
    dv16 = lax.rsqrt(da[...] + db[...] + 1.0)
    dv_o[...] = dv16
    dv1 = dv16[:, 0:1]
    glo_o[...] = jnp.concatenate([xt[...], xc[...]], axis=1) * dv1
    ghi_o[...] = jnp.concatenate([xd[...], xx[...]], axis=1) * dv1


_scale0 = pl.pallas_call(
    _scale0_body,
    grid=(GRID,),
    in_specs=[pl.BlockSpec((R, EMB), lambda i: (i, 0))] * 6,
    out_specs=[
        pl.BlockSpec((R, EMB), lambda i: (i, 0)),
        pl.BlockSpec((R, 32), lambda i: (i, 0)),
        pl.BlockSpec((R, 32), lambda i: (i, 0)),
    ],
    out_shape=[
        jax.ShapeDtypeStruct((N_PAD, EMB), jnp.float32),
        jax.ShapeDtypeStruct((N_PAD, 32), jnp.float32),
        jax.ShapeDtypeStruct((N_PAD, 32), jnp.float32),
    ],
)


def _layer1_body(mlo, mhi, glo, ghi, dv, W, b, olo, ohi):
    m = jnp.concatenate([mlo[...], mhi[...]], axis=1) + jnp.concatenate(
        [glo[...], ghi[...]], axis=1)
    dv1 = dv[...][:, 0:1]
    agg = m * dv1
    h = jnp.maximum(jnp.dot(agg, W[...], preferred_element_type=jnp.float32) + b[...], 0.0)
    g = h * dv1
    olo[...] = g[:, :32]
    ohi[...] = g[:, 32:]


_layer1 = pl.pallas_call(
    _layer1_body,
    grid=(GRID,),
    in_specs=[
        pl.BlockSpec((R, 32), lambda i: (i, 0)),
        pl.BlockSpec((R, 32), lambda i: (i, 0)),
        pl.BlockSpec((R, 32), lambda i: (i, 0)),
        pl.BlockSpec((R, 32), lambda i: (i, 0)),
        pl.BlockSpec((R, EMB), lambda i: (i, 0)),
        pl.BlockSpec((HID, HID), lambda i: (0, 0)),
        pl.BlockSpec((1, HID), lambda i: (0, 0)),
    ],
    out_specs=[
        pl.BlockSpec((R, 32), lambda i: (i, 0)),
        pl.BlockSpec((R, 32), lambda i: (i, 0)),
    ],
    out_shape=[
        jax.ShapeDtypeStruct((N_PAD, 32), jnp.float32),
        jax.ShapeDtypeStruct((N_PAD, 32), jnp.float32),
    ],
)


def _layer2_body(mlo, mhi, glo, ghi, dv, W, b, bat, P, pb, pooled, counts, out):
    i = pl.program_id(0)
    m = jnp.concatenate([mlo[...], mhi[...]], axis=1) + jnp.concatenate(
        [glo[...], ghi[...]], axis=1)
    dv1 = dv[...][:, 0:1]
    agg = m * dv1
    h = jnp.maximum(jnp.dot(agg, W[...], preferred_element_type=jnp.float32) + b[...], 0.0)
    brow = bat[...].reshape(1, R)
    oh = (lax.broadcasted_iota(jnp.int32, (B, R), 0) == brow).astype(jnp.float32)
    p = jnp.dot(oh, h, preferred_element_type=jnp.float32)
    cnt = jnp.dot(oh, jnp.ones((R, HID), jnp.float32), preferred_element_type=jnp.float32)

    @pl.when(i == 0)
    def _():
        pooled[...] = jnp.zeros_like(pooled)
        counts[...] = jnp.zeros_like(counts)

    pooled[...] += p
    counts[...] += cnt

    @pl.when(i == GRID - 1)
    def _():
        mean = pooled[...] / jnp.maximum(counts[...], 1.0)
        out[...] = jnp.dot(mean, P[...], preferred_element_type=jnp.float32) + pb[...]


_layer2 = pl.pallas_call(
    _layer2_body,
    grid=(GRID,),
    in_specs=[
        pl.BlockSpec((R, 32), lambda i: (i, 0)),
        pl.BlockSpec((R, 32), lambda i: (i, 0)),
        pl.BlockSpec((R, 32), lambda i: (i, 0)),
        pl.BlockSpec((R, 32), lambda i: (i, 0)),
        pl.BlockSpec((R, EMB), lambda i: (i, 0)),
        pl.BlockSpec((HID, HID), lambda i: (0, 0)),
        pl.BlockSpec((1, HID), lambda i: (0, 0)),
        pl.BlockSpec((1, 1, R), lambda i: (i, 0, 0)),
        pl.BlockSpec((HID, HID), lambda i: (0, 0)),
        pl.BlockSpec((1, HID), lambda i: (0, 0)),
    ],
    out_specs=[
        pl.BlockSpec((B, HID), lambda i: (0, 0)),
        pl.BlockSpec((B, HID), lambda i: (0, 0)),
        pl.BlockSpec((B, HID), lambda i: (0, 0)),
    ],
    out_shape=[
        jax.ShapeDtypeStruct((B, HID), jnp.float32),
        jax.ShapeDtypeStruct((B, HID), jnp.float32),
        jax.ShapeDtypeStruct((B, HID), jnp.float32),
    ],
)


def kernel(node_feats_raw, edge_index, batch_index, t_emb, c_emb, d_emb, x_emb,
           gcn_W1, gcn_b1, gcn_W2, gcn_b2, proj_W, proj_b):
    f = node_feats_raw
    t = jnp.maximum(f[:, 0] % T_BUCKETS, 0)
    c = jnp.maximum(f[:, 1] % C_BUCKETS, 0)
    dd = jnp.clip(f[:, 2], 0, 255)
    xx = jnp.clip(f[:, 3], 0, 7)

    def pad_n(a):
        return jnp.pad(a, (0, N_PAD - N))

    ti, ci, di, xi = pad_n(t), pad_n(c), pad_n(dd), pad_n(xx)
    src = jnp.pad(edge_index[0], (0, E_PAD - E), constant_values=N_PAD - 1)
    dst = jnp.pad(edge_index[1], (0, E_PAD - E), constant_values=N_PAD - 1)
    ones16 = jnp.ones((ECE, EMB), jnp.float32)
    zeros16 = jnp.zeros((NPT, EMB), jnp.float32)
    zeros32 = jnp.zeros((NPT, 32), jnp.float32)

    xt, xc, xd, xxe, dega, degb = _embdeg_kernel(
        ti, ci, di, xi, t_emb, c_emb, d_emb, x_emb, dst, ones16, zeros16)
    dv16, g0lo, g0hi = _scale0(dega, degb, xt, xc, xd, xxe)
    m1lo, m1hi = _agg_kernel(g0lo, g0hi, src, dst, zeros32)
    g1lo, g1hi = _layer1(m1lo, m1hi, g0lo, g0hi, dv16, gcn_W1,
                         gcn_b1.reshape(1, HID))
    m2lo, m2hi = _agg_kernel(g1lo, g1hi, src, dst, zeros32)
    bat = jnp.pad(batch_index, (0, N_PAD - N), constant_values=B).reshape(GRID, 1, R)
    pooled, counts, out = _layer2(m2lo, m2hi, g1lo, g1hi, dv16, gcn_W2,
                                  gcn_b2.reshape(1, HID), bat, proj_W,
                                  proj_b.reshape(1, HID))
    return out


# speedup vs baseline: 1.0277x; 1.0277x over previous
import functools

import jax
import jax.numpy as jnp
from jax import lax
from jax.experimental import pallas as pl
from jax.experimental.pallas import tpu as pltpu
from jax.experimental.pallas import tpu_sc as plsc

N = 50000
E = 800000
HID = 64
EMB = 16
B = 32
T_BUCKETS = 64
C_BUCKETS = 2048

NC = 2
NS = 16
NW = NC * NS

N_PAD = 50176
NPW = N_PAD // NW
NPT = N_PAD // NS
EC = 112
ECE = 416
NCHE = 124
EPT = ECE * NCHE
E_PAD = EPT * NS
EPW = E_PAD // NW
R = 512
GRID = N_PAD // R


def _mesh():
    return plsc.VectorSubcoreMesh(core_axis_name="c", subcore_axis_name="s")


_SC_PARAMS = pltpu.CompilerParams(use_tc_tiling_on_sc=False)


@functools.partial(
    pl.kernel,
    out_type=[jax.ShapeDtypeStruct((N_PAD, EMB), jnp.float32)] * 4,
    mesh=_mesh(),
    compiler_params=_SC_PARAMS,
    scratch_types=(
        [pltpu.VMEM((EC,), jnp.int32)] * 4
        + [pltpu.VMEM((EC, EMB), jnp.float32)] * 4
        + [pltpu.SemaphoreType.DMA] * 4
    ),
)
def _emb_kernel(ti, ci, di, xi, te, ce, de, xe, ot, oc, od, ox,
                i0, i1, i2, i3, r0, r1, r2, r3, m0, m1, m2, m3):
    wid = lax.axis_index("c") * NS + lax.axis_index("s")
    base = wid * NPW
    lanes = ((ti, te, ot, i0, r0, m0), (ci, ce, oc, i1, r1, m1),
             (di, de, od, i2, r2, m2), (xi, xe, ox, i3, r3, m3))

    def chunk(j, carry):
        off = base + j * EC
        for iref, tab, out, iv, rv, sem in lanes:
            pltpu.sync_copy(iref.at[pl.ds(off, EC)], iv)
            pltpu.async_copy(tab.at[iv], rv, sem)
        for iref, tab, out, iv, rv, sem in lanes:
            pltpu.make_async_copy(tab.at[iv], rv, sem).wait()
            pltpu.sync_copy(rv, out.at[pl.ds(off, EC)])
        return carry

    lax.fori_loop(0, NPW // EC, chunk, 0)


@functools.partial(
    pl.kernel,
    out_type=[jax.ShapeDtypeStruct((N_PAD, EMB), jnp.float32)] * 2,
    mesh=_mesh(),
    compiler_params=_SC_PARAMS,
    scratch_types=[
        pltpu.VMEM((ECE,), jnp.int32),
        pltpu.VMEM((ECE, EMB), jnp.float32),
        pltpu.VMEM_SHARED((N_PAD, EMB), jnp.float32),
    ],
)
def _deg_kernel(dst, ones_h, zeros_h, dega, degb, idx_v, ones_v, acc):
    c = lax.axis_index("c")
    s = lax.axis_index("s")
    pltpu.sync_copy(zeros_h, acc.at[pl.ds(s * NPT, NPT)])
    pltpu.sync_copy(ones_h, ones_v)
    plsc.subcore_barrier()
    ebase = (c * NS + s) * EPW

    def chunk(j, carry):
        off = ebase + j * ECE
        pltpu.sync_copy(dst.at[pl.ds(off, ECE)], idx_v)
        pltpu.sync_copy(ones_v, acc.at[idx_v], add=True)
        return carry

    lax.fori_loop(0, EPW // ECE, chunk, 0)
    plsc.subcore_barrier()
    nb = s * NPT

    @pl.when(c == 0)
    def _():
        pltpu.sync_copy(acc.at[pl.ds(nb, NPT)], dega.at[pl.ds(nb, NPT)])

    @pl.when(c == 1)
    def _():
        pltpu.sync_copy(acc.at[pl.ds(nb, NPT)], degb.at[pl.ds(nb, NPT)])


@functools.partial(
    pl.kernel,
    out_type=[jax.ShapeDtypeStruct((N_PAD, 32), jnp.float32)] * 2,
    mesh=_mesh(),
    compiler_params=_SC_PARAMS,
    scratch_types=(
        [pltpu.VMEM((ECE,), jnp.int32)] * 4
        + [pltpu.VMEM((ECE, 32), jnp.float32)] * 2
        + [pltpu.VMEM_SHARED((N_PAD, 32), jnp.float32)]
        + [pltpu.SemaphoreType.DMA] * 2
    ),
)
def _agg_kernel(glo, ghi, src, dst, zeros_h, mlo, mhi,
                s0, s1, d0, d1, r0, r1, acc, semA, semB):
    c = lax.axis_index("c")
    s = lax.axis_index("s")
    pltpu.sync_copy(zeros_h, acc.at[pl.ds(s * NPT, NPT)])
    plsc.subcore_barrier()
    ebase = s * EPT

    def run(g):
        bufs = ((s0, d0, r0, semA), (s1, d1, r1, semB))

        def fire(bi, j):
            sv, dv, rv, sem = bufs[bi]
            off = ebase + j * ECE
            pltpu.sync_copy(src.at[pl.ds(off, ECE)], sv)
            pltpu.sync_copy(dst.at[pl.ds(off, ECE)], dv)
            pltpu.async_copy(g.at[sv], rv, sem)

        def drain_scatter(bi):
            sv, dv, rv, sem = bufs[bi]
            pltpu.make_async_copy(g.at[sv], rv, sem).wait()
            pltpu.sync_copy(rv, acc.at[dv], add=True)

        fire(0, 0)

        def body(jj, carry):
            j0 = jj * 2
            for b in (0, 1):
                nxt = j0 + b + 1

                @pl.when(nxt < NCHE)
                def _():
                    fire((b + 1) % 2, nxt)

                drain_scatter(b)
            return carry

        lax.fori_loop(0, NCHE // 2, body, 0)

    @pl.when(c == 0)
    def _():
        run(glo)

    @pl.when(c == 1)
    def _():
        run(ghi)

    plsc.subcore_barrier()
    nb = s * NPT

    @pl.when(c == 0)
    def _():
        pltpu.sync_copy(acc.at[pl.ds(nb, NPT)], mlo.at[pl.ds(nb, NPT)])

    @pl.when(c == 1)
    def _():
        pltpu.sync_copy(acc.at[pl.ds(nb, NPT)], mhi.at[pl.ds(nb, NPT)])


def _scale0_body(da, db, xt, xc, xd, xx, dv_o, glo_o, ghi_o):
    dv16 = lax.rsqrt(da[...] + db[...] + 1.0)
    dv_o[...] = dv16
    dv1 = dv16[:, 0:1]
    glo_o[...] = jnp.concatenate([xt[...], xc[...]], axis=1) * dv1
    ghi_o[...] = jnp.concatenate([xd[...], xx[...]], axis=1) * dv1


_scale0 = pl.pallas_call(
    _scale0_body,
    grid=(GRID,),
    in_specs=[pl.BlockSpec((R, EMB), lambda i: (i, 0))] * 6,
    out_specs=[
        pl.BlockSpec((R, EMB), lambda i: (i, 0)),
        pl.BlockSpec((R, 32), lambda i: (i, 0)),
        pl.BlockSpec((R, 32), lambda i: (i, 0)),
    ],
    out_shape=[
        jax.ShapeDtypeStruct((N_PAD, EMB), jnp.float32),
        jax.ShapeDtypeStruct((N_PAD, 32), jnp.float32),
        jax.ShapeDtypeStruct((N_PAD, 32), jnp.float32),
    ],
)


def _layer1_body(mlo, mhi, glo, ghi, dv, W, b, olo, ohi):
    m = jnp.concatenate([mlo[...], mhi[...]], axis=1) + jnp.concatenate(
        [glo[...], ghi[...]], axis=1)
    dv1 = dv[...][:, 0:1]
    agg = m * dv1
    h = jnp.maximum(jnp.dot(agg, W[...], preferred_element_type=jnp.float32) + b[...], 0.0)
    g = h * dv1
    olo[...] = g[:, :32]
    ohi[...] = g[:, 32:]


_layer1 = pl.pallas_call(
    _layer1_body,
    grid=(GRID,),
    in_specs=[
        pl.BlockSpec((R, 32), lambda i: (i, 0)),
        pl.BlockSpec((R, 32), lambda i: (i, 0)),
        pl.BlockSpec((R, 32), lambda i: (i, 0)),
        pl.BlockSpec((R, 32), lambda i: (i, 0)),
        pl.BlockSpec((R, EMB), lambda i: (i, 0)),
        pl.BlockSpec((HID, HID), lambda i: (0, 0)),
        pl.BlockSpec((1, HID), lambda i: (0, 0)),
    ],
    out_specs=[
        pl.BlockSpec((R, 32), lambda i: (i, 0)),
        pl.BlockSpec((R, 32), lambda i: (i, 0)),
    ],
    out_shape=[
        jax.ShapeDtypeStruct((N_PAD, 32), jnp.float32),
        jax.ShapeDtypeStruct((N_PAD, 32), jnp.float32),
    ],
)


def _layer2_body(mlo, mhi, glo, ghi, dv, W, b, bat, P, pb, pooled, counts, out):
    i = pl.program_id(0)
    m = jnp.concatenate([mlo[...], mhi[...]], axis=1) + jnp.concatenate(
        [glo[...], ghi[...]], axis=1)
    dv1 = dv[...][:, 0:1]
    agg = m * dv1
    h = jnp.maximum(jnp.dot(agg, W[...], preferred_element_type=jnp.float32) + b[...], 0.0)
    brow = bat[...].reshape(1, R)
    oh = (lax.broadcasted_iota(jnp.int32, (B, R), 0) == brow).astype(jnp.float32)
    p = jnp.dot(oh, h, preferred_element_type=jnp.float32)
    cnt = jnp.dot(oh, jnp.ones((R, HID), jnp.float32), preferred_element_type=jnp.float32)

    @pl.when(i == 0)
    def _():
        pooled[...] = jnp.zeros_like(pooled)
        counts[...] = jnp.zeros_like(counts)

    pooled[...] += p
    counts[...] += cnt

    @pl.when(i == GRID - 1)
    def _():
        mean = pooled[...] / jnp.maximum(counts[...], 1.0)
        out[...] = jnp.dot(mean, P[...], preferred_element_type=jnp.float32) + pb[...]


_layer2 = pl.pallas_call(
    _layer2_body,
    grid=(GRID,),
    in_specs=[
        pl.BlockSpec((R, 32), lambda i: (i, 0)),
        pl.BlockSpec((R, 32), lambda i: (i, 0)),
        pl.BlockSpec((R, 32), lambda i: (i, 0)),
        pl.BlockSpec((R, 32), lambda i: (i, 0)),
        pl.BlockSpec((R, EMB), lambda i: (i, 0)),
        pl.BlockSpec((HID, HID), lambda i: (0, 0)),
        pl.BlockSpec((1, HID), lambda i: (0, 0)),
        pl.BlockSpec((1, 1, R), lambda i: (i, 0, 0)),
        pl.BlockSpec((HID, HID), lambda i: (0, 0)),
        pl.BlockSpec((1, HID), lambda i: (0, 0)),
    ],
    out_specs=[
        pl.BlockSpec((B, HID), lambda i: (0, 0)),
        pl.BlockSpec((B, HID), lambda i: (0, 0)),
        pl.BlockSpec((B, HID), lambda i: (0, 0)),
    ],
    out_shape=[
        jax.ShapeDtypeStruct((B, HID), jnp.float32),
        jax.ShapeDtypeStruct((B, HID), jnp.float32),
        jax.ShapeDtypeStruct((B, HID), jnp.float32),
    ],
)


def kernel(node_feats_raw, edge_index, batch_index, t_emb, c_emb, d_emb, x_emb,
           gcn_W1, gcn_b1, gcn_W2, gcn_b2, proj_W, proj_b):
    f = node_feats_raw
    t = jnp.maximum(f[:, 0] % T_BUCKETS, 0)
    c = jnp.maximum(f[:, 1] % C_BUCKETS, 0)
    dd = jnp.clip(f[:, 2], 0, 255)
    xx = jnp.clip(f[:, 3], 0, 7)

    def pad_n(a):
        return jnp.pad(a, (0, N_PAD - N))

    ti, ci, di, xi = pad_n(t), pad_n(c), pad_n(dd), pad_n(xx)
    src = jnp.pad(edge_index[0], (0, E_PAD - E), constant_values=N_PAD - 1)
    dst = jnp.pad(edge_index[1], (0, E_PAD - E), constant_values=N_PAD - 1)
    ones16 = jnp.ones((ECE, EMB), jnp.float32)
    zeros16 = jnp.zeros((NPT, EMB), jnp.float32)
    zeros32 = jnp.zeros((NPT, 32), jnp.float32)

    xt, xc, xd, xxe = _emb_kernel(ti, ci, di, xi, t_emb, c_emb, d_emb, x_emb)
    dega, degb = _deg_kernel(dst, ones16, zeros16)
    dv16, g0lo, g0hi = _scale0(dega, degb, xt, xc, xd, xxe)
    m1lo, m1hi = _agg_kernel(g0lo, g0hi, src, dst, zeros32)
    g1lo, g1hi = _layer1(m1lo, m1hi, g0lo, g0hi, dv16, gcn_W1,
                         gcn_b1.reshape(1, HID))
    m2lo, m2hi = _agg_kernel(g1lo, g1hi, src, dst, zeros32)
    bat = jnp.pad(batch_index, (0, N_PAD - N), constant_values=B).reshape(GRID, 1, R)
    pooled, counts, out = _layer2(m2lo, m2hi, g1lo, g1hi, dv16, gcn_W2,
                                  gcn_b2.reshape(1, HID), bat, proj_W,
                                  proj_b.reshape(1, HID))
    return out
